# Initial kernel scaffold; baseline (speedup 1.0000x reference)
import jax
import jax.numpy as jnp
from jax.experimental import pallas as pl


def kernel(positions, node_attrs, edge_index, shifts, cell, batch, ptr, params):
    raise NotImplementedError("write your pallas kernel here")


# speedup vs baseline: 2.7493x; 2.7493x over previous
import functools
import numpy as np
import jax
import jax.numpy as jnp
from jax import lax
from jax.experimental import pallas as pl
from jax.experimental.pallas import tpu as pltpu
from jax.experimental.pallas import tpu_sc as plsc

C = 32
NB = 8
SH = 9
R_MAX = 5.0
AVG = 32.0
S3 = float(np.sqrt(3.0))
S5 = float(np.sqrt(5.0))
S15 = float(np.sqrt(15.0))
PREF = float(np.sqrt(2.0 / R_MAX))
KN = (np.arange(1, NB + 1, dtype=np.float32) * np.pi / R_MAX).reshape(1, NB)

BE = 1280
BN = 2000

NW = 32
KCH = 80


def _silu(x):
    return x * jax.nn.sigmoid(x)


def _dsilu(x):
    s = jax.nn.sigmoid(x)
    return s * (1.0 + x * (1.0 - s))


def _dot(a, b):
    return jnp.dot(a, b, preferred_element_type=jnp.float32,
                   precision=lax.Precision.HIGHEST)


def _dotd(a, b):
    return jnp.dot(a, b, preferred_element_type=jnp.float32)


def _sc_gather(table, idx):
    n, d = table.shape
    e = idx.shape[0]
    per_w = e // NW
    iters = per_w // KCH
    mesh = plsc.VectorSubcoreMesh(core_axis_name="c", subcore_axis_name="s")

    @functools.partial(
        pl.kernel, mesh=mesh,
        compiler_params=pltpu.CompilerParams(use_tc_tiling_on_sc=False),
        out_type=jax.ShapeDtypeStruct((e, d), jnp.float32),
        scratch_types=[
            pltpu.VMEM((KCH,), jnp.int32),
            pltpu.VMEM((KCH, d), jnp.float32),
            pltpu.SemaphoreType.DMA,
        ],
    )
    def k(tab, ix, out, idx_v, rows_v, sem):
        cid = lax.axis_index("c")
        sid = lax.axis_index("s")
        wid = sid * 2 + cid
        base = wid * per_w

        def body(j, carry):
            off = base + j * KCH
            pltpu.sync_copy(ix.at[pl.ds(off, KCH)], idx_v)
            pltpu.async_copy(tab.at[idx_v], rows_v, sem).wait()
            pltpu.sync_copy(rows_v, out.at[pl.ds(off, KCH)])
            return carry

        lax.fori_loop(0, iters, body, 0)

    return k(table, idx)


def _sc_scatter_add(payload, idx, n, zeros_nd):
    e, d = payload.shape
    per_w = e // NW
    iters = per_w // KCH
    rows_per_tile = n // 16
    mesh = plsc.VectorSubcoreMesh(core_axis_name="c", subcore_axis_name="s")

    @functools.partial(
        pl.kernel, mesh=mesh,
        compiler_params=pltpu.CompilerParams(use_tc_tiling_on_sc=False),
        out_type=jax.ShapeDtypeStruct((2, n, d), jnp.float32),
        scratch_types=[
            pltpu.VMEM((1, KCH), jnp.int32),
            pltpu.VMEM((KCH, d), jnp.float32),
            pltpu.VMEM_SHARED((n, d), jnp.float32),
        ],
    )
    def k(pay, ix, zr, out, idx_v, pay_v, acc):
        cid = lax.axis_index("c")
        sid = lax.axis_index("s")
        wid = sid * 2 + cid
        base = wid * per_w
        r0 = sid * rows_per_tile
        pltpu.sync_copy(zr.at[pl.ds(r0, rows_per_tile)],
                        acc.at[pl.ds(r0, rows_per_tile)])
        plsc.subcore_barrier()

        def body(j, carry):
            off = base + j * KCH
            row = wid * iters + j
            pltpu.sync_copy(ix.at[pl.ds(row, 1)], idx_v)
            pltpu.sync_copy(pay.at[pl.ds(off, KCH)], pay_v)
            pltpu.sync_copy(pay_v, acc.at[idx_v.at[0]], add=True)
            return carry

        lax.fori_loop(0, iters, body, 0)
        plsc.subcore_barrier()
        pltpu.sync_copy(acc.at[pl.ds(r0, rows_per_tile)],
                        out.at[cid, pl.ds(r0, rows_per_tile)])

    return k(payload, idx.reshape(e // KCH, KCH), zeros_nd)


def _mlp_fwd(f, w1, w2, w3):
    r1 = _dotd(f, w1)
    a1 = _silu(r1)
    r2 = _dotd(a1, w2)
    a2 = _silu(r2)
    return r1, r2, _dotd(a2, w3)


def _mlp_bwd(gr, f, w1, w2, r1, r2, w3t, w2t, w1t):
    ga2 = _dotd(gr, w3t)
    gr2 = ga2 * _dsilu(r2)
    ga1 = _dotd(gr2, w2t)
    gr1 = ga1 * _dsilu(r1)
    return _dotd(gr1, w1t)


def _geom_edge0_body(ps_ref, pd_ref, w1, w2, w3, wm, we, wv,
                     geo_ref, scat_ref):
    ps = ps_ref[...]
    pd = pd_ref[...]
    v = pd[:, 0:3] - ps[:, 0:3]
    ln = jnp.sqrt(jnp.sum(v * v, axis=1, keepdims=True) + 1e-12)
    u = v / ln
    x = u[:, 0:1]; y = u[:, 1:2]; z = u[:, 2:3]
    one = jnp.ones_like(x)
    a = jnp.concatenate([
        one, S3 * x, S3 * y, S3 * z,
        S15 * x * y, S15 * y * z,
        0.5 * S5 * (2.0 * z * z - x * x - y * y),
        S15 * x * z, 0.5 * S15 * (x * x - y * y)], axis=1)
    kn = (lax.broadcasted_iota(jnp.int32, (1, NB), 1).astype(jnp.float32) + 1.0) * (np.pi / R_MAX)
    sin_t = jnp.sin(ln * kn)
    bes = PREF * sin_t / ln
    uu = ln / R_MAX
    msk = (uu < 1.0).astype(jnp.float32)
    u4 = uu * uu * uu * uu * uu
    cut = (1.0 - 21.0 * u4 + 35.0 * u4 * uu - 15.0 * u4 * uu * uu) * msk
    f = bes * cut
    pad = jnp.zeros((ps.shape[0], 11), jnp.float32)
    geo_ref[...] = jnp.concatenate([u, ln, a, f, pad], axis=1)

    _, _, r = _mlp_fwd(f, w1[...], w2[...], w3[...])
    msg = r * we[...]
    acc = jnp.zeros((ps.shape[0], C), jnp.float32)
    wmv = wm[...]
    for s in range(SH):
        acc = acc + _dot(msg * a[:, s:s + 1], wmv[s])
    vc = _dot(msg, wv[...]) * a[:, 1:4]
    pad1 = jnp.zeros((ps.shape[0], 5), jnp.float32)
    scat_ref[...] = jnp.concatenate([acc, vc, pad1], axis=1)


def _node0_body(agg_ref, we, wsc, wp, wpost, wread, h1_ref, m0_ref, ev_ref):
    p = agg_ref[0] + agg_ref[1]
    m0 = p[:, 0:C] * (1.0 / AVG)
    wpv = wp[...]
    w0 = wpv[0:1, :]; w1 = wpv[1:2, :]; w2 = wpv[2:3, :]
    feats = w0 * m0 + w1 * m0 * m0 + w2 * m0 * m0 * m0
    scrow = _dotd(we[...], wsc[...])
    h1 = _dotd(feats, wpost[...]) + scrow
    h1_ref[...] = h1
    m0_ref[...] = m0
    en = _dotd(h1, wread[...])
    ev_ref[...] = jnp.concatenate([en, p[:, C:C + 3]], axis=1)


def _edge1_body(geo_ref, g1_ref, w1, w2, w3, wm, wv, scat_ref):
    geo = geo_ref[...]
    a = geo[:, 4:13]
    f = geo[:, 13:21]
    _, _, r = _mlp_fwd(f, w1[...], w2[...], w3[...])
    g1 = g1_ref[...]
    msg = g1 * r
    acc = jnp.zeros((geo.shape[0], C), jnp.float32)
    wmv = wm[...]
    for s in range(SH):
        acc = acc + _dot(msg * a[:, s:s + 1], wmv[s])
    vc = _dot(msg, wv[...]) * a[:, 1:4]
    pad1 = jnp.zeros((geo.shape[0], 5), jnp.float32)
    scat_ref[...] = jnp.concatenate([acc, vc, pad1], axis=1)


def _node1_body(agg_ref, h1_ref, ev_ref, wsc, wp, wpost, wmlp1, wmlp2t,
                wsct, wpostt, wmlp1t, wreadt,
                gm1_ref, gh1a_ref, en_ref, vec_ref):
    p = agg_ref[0] + agg_ref[1]
    h1 = h1_ref[...]
    ev = ev_ref[...]
    m1 = p[:, 0:C] * (1.0 / AVG)
    wpv = wp[...]
    w0 = wpv[0:1, :]; w1 = wpv[1:2, :]; w2 = wpv[2:3, :]
    feats = w0 * m1 + w1 * m1 * m1 + w2 * m1 * m1 * m1
    h2 = _dotd(feats, wpost[...]) + _dotd(h1, wsc[...])
    u2 = _dotd(h2, wmlp1[...])
    en_b = _dotd(_silu(u2), jnp.transpose(wmlp2t[...]))
    en_ref[...] = ev[:, 0:1] + en_b
    vec_ref[...] = (ev[:, 1:4] + p[:, C:C + 3]) * (1.0 / AVG)
    gh2 = _dotd(_dsilu(u2) * wmlp2t[...], wmlp1t[...])
    gfeats = _dotd(gh2, wpostt[...])
    gm1 = gfeats * (w0 + 2.0 * w1 * m1 + 3.0 * w2 * m1 * m1)
    gm1_ref[...] = gm1 * (1.0 / AVG)
    gh1a_ref[...] = _dotd(gh2, wsct[...]) + wreadt[...]


def _edge1_bwd_body(geo_ref, g1_ref, gg1_ref, w1, w2, w3, wmt,
                    w3t, w2t, w1t, gsrc_ref, gaf_ref):
    geo = geo_ref[...]
    a = geo[:, 4:13]
    f = geo[:, 13:21]
    w1v = w1[...]; w2v = w2[...]
    r1, r2, r = _mlp_fwd(f, w1v, w2v, w3[...])
    g1 = g1_ref[...]
    msg = g1 * r
    gg1 = gg1_ref[...]
    wmtv = wmt[...]
    gmsg = jnp.zeros((geo.shape[0], C), jnp.float32)
    gas = []
    for s in range(SH):
        ys = _dot(gg1, wmtv[s])
        gmsg = gmsg + a[:, s:s + 1] * ys
        gas.append(jnp.sum(msg * ys, axis=1, keepdims=True))
    ga = jnp.concatenate(gas, axis=1)
    gsrc_ref[...] = gmsg * r
    gr = gmsg * g1
    gf = _mlp_bwd(gr, f, w1v, w2v, r1, r2, w3t[...], w2t[...], w1t[...])
    pad = jnp.zeros((geo.shape[0], 7), jnp.float32)
    gaf_ref[...] = jnp.concatenate([ga, gf, pad], axis=1)


def _node0_bwd_body(gh1a_ref, ghb_ref, m0_ref, wpostt, wp, gm0_ref):
    gh1 = gh1a_ref[...] + ghb_ref[0] + ghb_ref[1]
    gfeats = _dotd(gh1, wpostt[...])
    m0 = m0_ref[...]
    wpv = wp[...]
    w0 = wpv[0:1, :]; w1 = wpv[1:2, :]; w2 = wpv[2:3, :]
    gm0 = gfeats * (w0 + 2.0 * w1 * m0 + 3.0 * w2 * m0 * m0)
    gm0_ref[...] = gm0 * (1.0 / AVG)


def _edge0_bwd_body(geo_ref, gg0_ref, gaf_ref, w1, w2, w3, wmt, we,
                    w3t, w2t, w1t, gv_ref):
    geo = geo_ref[...]
    u = geo[:, 0:3]
    ln = geo[:, 3:4]
    a = geo[:, 4:13]
    f = geo[:, 13:21]
    w1v = w1[...]; w2v = w2[...]
    r1, r2, r = _mlp_fwd(f, w1v, w2v, w3[...])
    msg = r * we[...]
    gg0 = gg0_ref[...]
    gaf = gaf_ref[...]
    wmtv = wmt[...]
    gmsg = jnp.zeros((geo.shape[0], C), jnp.float32)
    gas = []
    for s in range(SH):
        ys = _dot(gg0, wmtv[s])
        gmsg = gmsg + a[:, s:s + 1] * ys
        gas.append(jnp.sum(msg * ys, axis=1, keepdims=True))
    ga = jnp.concatenate(gas, axis=1) + gaf[:, 0:SH]
    gr = gmsg * we[...]
    gf = _mlp_bwd(gr, f, w1v, w2v, r1, r2, w3t[...], w2t[...], w1t[...])
    gf = gf + gaf[:, SH:SH + NB]
    kn = (lax.broadcasted_iota(jnp.int32, (1, NB), 1).astype(jnp.float32) + 1.0) * (np.pi / R_MAX)
    sin_t = jnp.sin(ln * kn)
    cos_t = jnp.cos(ln * kn)
    bes = PREF * sin_t / ln
    uu = ln / R_MAX
    msk = (uu < 1.0).astype(jnp.float32)
    u4 = uu * uu * uu * uu
    u5 = u4 * uu
    cut = (1.0 - 21.0 * u5 + 35.0 * u5 * uu - 15.0 * u5 * uu * uu) * msk
    dbes = PREF * (kn * cos_t / ln - sin_t / (ln * ln))
    dcut = (-105.0 * u4 + 210.0 * u5 - 105.0 * u5 * uu) * msk * (1.0 / R_MAX)
    glen = jnp.sum(gf * (dbes * cut + bes * dcut), axis=1, keepdims=True)
    x = u[:, 0:1]; y = u[:, 1:2]; z = u[:, 2:3]
    ga1 = ga[:, 1:2]; ga2 = ga[:, 2:3]; ga3 = ga[:, 3:4]
    ga4 = ga[:, 4:5]; ga5 = ga[:, 5:6]; ga6 = ga[:, 6:7]
    ga7 = ga[:, 7:8]; ga8 = ga[:, 8:9]
    gx = S3 * ga1 + S15 * (y * ga4 + z * ga7) - S5 * x * ga6 + S15 * x * ga8
    gy = S3 * ga2 + S15 * (x * ga4 + z * ga5) - S5 * y * ga6 - S15 * y * ga8
    gz = S3 * ga3 + S15 * (y * ga5 + x * ga7) + 2.0 * S5 * z * ga6
    gu = jnp.concatenate([gx, gy, gz], axis=1)
    gv = glen * u + (gu - u * jnp.sum(u * gu, axis=1, keepdims=True)) / ln
    pad1 = jnp.zeros((geo.shape[0], 5), jnp.float32)
    gv_ref[...] = jnp.concatenate([gv, pad1], axis=1)


def _forces_body(fs_ref, fd_ref, out_ref):
    out_ref[...] = (fs_ref[0] + fs_ref[1] - fd_ref[0] - fd_ref[1])[:, 0:3]


def _energy_body(en_ref, na_ref, ae_ref, out_ref):
    out_ref[...] = jnp.sum(en_ref[...] + na_ref[...] * ae_ref[0, 0],
                           axis=1, keepdims=True)


def _wspec(shape):
    nd = len(shape)
    return pl.BlockSpec(shape, lambda i, _nd=nd: (0,) * _nd)


def kernel(positions, node_attrs, edge_index, shifts, cell, batch, ptr, params):
    n = positions.shape[0]
    e = edge_index.shape[1]
    g = cell.shape[0]
    ge = e // BE
    gn = n // BN
    f32 = jnp.float32

    src = edge_index[0].astype(jnp.int32)
    dst = edge_index[1].astype(jnp.int32)
    pos8 = jnp.concatenate([positions.astype(f32),
                            jnp.zeros((n, 5), f32)], axis=1)

    p = {k: v.astype(f32) for k, v in params.items()}
    wm = [p['W_msg_%d' % i].reshape(C, SH, C).transpose(1, 0, 2)
          for i in range(2)]
    wmt = [jnp.transpose(w, (0, 2, 1)) for w in wm]
    z40 = jnp.zeros((n, 40), f32)
    z32 = jnp.zeros((n, 32), f32)
    z8 = jnp.zeros((n, 8), f32)

    espec = lambda d: pl.BlockSpec((BE, d), lambda i: (i, 0))
    nspec = lambda d: pl.BlockSpec((BN, d), lambda i: (i, 0))
    n2spec = lambda d: pl.BlockSpec((2, BN, d), lambda i: (0, i, 0))

    ps = _sc_gather(pos8, src)
    pd = _sc_gather(pos8, dst)

    geo, scat0 = pl.pallas_call(
        _geom_edge0_body,
        grid=(ge,),
        in_specs=[espec(8), espec(8), _wspec((NB, 64)), _wspec((64, 64)),
                  _wspec((64, C)), _wspec((SH, C, C)), _wspec((1, C)),
                  _wspec((C, 1))],
        out_specs=[espec(32), espec(40)],
        out_shape=[jax.ShapeDtypeStruct((e, 32), f32),
                   jax.ShapeDtypeStruct((e, 40), f32)],
    )(ps, pd, p['W_r1_0'], p['W_r2_0'], p['W_r3_0'], wm[0],
      p['W_embed'], p['W_vec_0'])

    agg0 = _sc_scatter_add(scat0, dst, n, z40)

    h1, m0, ev0 = pl.pallas_call(
        _node0_body,
        grid=(gn,),
        in_specs=[n2spec(40), _wspec((1, C)), _wspec((C, C)),
                  _wspec((3, C)), _wspec((C, C)), _wspec((C, 1))],
        out_specs=[nspec(32), nspec(32), nspec(4)],
        out_shape=[jax.ShapeDtypeStruct((n, 32), f32),
                   jax.ShapeDtypeStruct((n, 32), f32),
                   jax.ShapeDtypeStruct((n, 4), f32)],
    )(agg0, p['W_embed'], p['W_sc_0'], p['W_prod_0'], p['W_post_0'],
      p['W_read_0'])

    g1 = _sc_gather(h1, src)

    scat1 = pl.pallas_call(
        _edge1_body,
        grid=(ge,),
        in_specs=[espec(32), espec(32), _wspec((NB, 64)), _wspec((64, 64)),
                  _wspec((64, C)), _wspec((SH, C, C)), _wspec((C, 1))],
        out_specs=espec(40),
        out_shape=jax.ShapeDtypeStruct((e, 40), f32),
    )(geo, g1, p['W_r1_1'], p['W_r2_1'], p['W_r3_1'], wm[1], p['W_vec_1'])

    agg1 = _sc_scatter_add(scat1, dst, n, z40)

    gm1, gh1a, en, vec_out = pl.pallas_call(
        _node1_body,
        grid=(gn,),
        in_specs=[n2spec(40), nspec(32), nspec(4), _wspec((C, C)),
                  _wspec((3, C)), _wspec((C, C)), _wspec((C, 16)),
                  _wspec((1, 16)), _wspec((C, C)), _wspec((C, C)),
                  _wspec((16, C)), _wspec((1, C))],
        out_specs=[nspec(32), nspec(32), nspec(1), nspec(3)],
        out_shape=[jax.ShapeDtypeStruct((n, 32), f32),
                   jax.ShapeDtypeStruct((n, 32), f32),
                   jax.ShapeDtypeStruct((n, 1), f32),
                   jax.ShapeDtypeStruct((n, 3), f32)],
    )(agg1, h1, ev0, p['W_sc_1'], p['W_prod_1'], p['W_post_1'], p['W_mlp1'],
      p['W_mlp2'].T, p['W_sc_1'].T, p['W_post_1'].T, p['W_mlp1'].T,
      p['W_read_0'].T)

    gg1 = _sc_gather(gm1, dst)

    gsrc, gaf1 = pl.pallas_call(
        _edge1_bwd_body,
        grid=(ge,),
        in_specs=[espec(32), espec(32), espec(32), _wspec((NB, 64)),
                  _wspec((64, 64)), _wspec((64, C)), _wspec((SH, C, C)),
                  _wspec((C, 64)), _wspec((64, 64)), _wspec((64, NB))],
        out_specs=[espec(32), espec(24)],
        out_shape=[jax.ShapeDtypeStruct((e, 32), f32),
                   jax.ShapeDtypeStruct((e, 24), f32)],
    )(geo, g1, gg1, p['W_r1_1'], p['W_r2_1'], p['W_r3_1'], wmt[1],
      p['W_r3_1'].T, p['W_r2_1'].T, p['W_r1_1'].T)

    ghb = _sc_scatter_add(gsrc, src, n, z32)

    gm0 = pl.pallas_call(
        _node0_bwd_body,
        grid=(gn,),
        in_specs=[nspec(32), n2spec(32), nspec(32), _wspec((C, C)),
                  _wspec((3, C))],
        out_specs=nspec(32),
        out_shape=jax.ShapeDtypeStruct((n, 32), f32),
    )(gh1a, ghb, m0, p['W_post_0'].T, p['W_prod_0'])

    gg0 = _sc_gather(gm0, dst)

    gv = pl.pallas_call(
        _edge0_bwd_body,
        grid=(ge,),
        in_specs=[espec(32), espec(32), espec(24), _wspec((NB, 64)),
                  _wspec((64, 64)), _wspec((64, C)), _wspec((SH, C, C)),
                  _wspec((1, C)), _wspec((C, 64)), _wspec((64, 64)),
                  _wspec((64, NB))],
        out_specs=espec(8),
        out_shape=jax.ShapeDtypeStruct((e, 8), f32),
    )(geo, gg0, gaf1, p['W_r1_0'], p['W_r2_0'], p['W_r3_0'], wmt[0],
      p['W_embed'], p['W_r3_0'].T, p['W_r2_0'].T, p['W_r1_0'].T)

    fs = _sc_scatter_add(gv, src, n, z8)
    fd = _sc_scatter_add(gv, dst, n, z8)

    forces = pl.pallas_call(
        _forces_body,
        grid=(gn,),
        in_specs=[n2spec(8), n2spec(8)],
        out_specs=nspec(3),
        out_shape=jax.ShapeDtypeStruct((n, 3), f32),
    )(fs, fd)

    npg = n // g
    te = pl.pallas_call(
        _energy_body,
        grid=(1,),
        in_specs=[pl.BlockSpec((g, npg), lambda i: (0, 0)),
                  pl.BlockSpec((g, npg), lambda i: (0, 0)),
                  pl.BlockSpec((1, 1), lambda i: (0, 0))],
        out_specs=pl.BlockSpec((g, 1), lambda i: (0, 0)),
        out_shape=jax.ShapeDtypeStruct((g, 1), f32),
    )(en.reshape(g, npg), node_attrs.astype(f32).reshape(g, npg),
      p['atomic_energies'].reshape(1, 1))

    return te[:, 0], forces, vec_out


# speedup vs baseline: 2.8120x; 1.0228x over previous
import functools
import numpy as np
import jax
import jax.numpy as jnp
from jax import lax
from jax.experimental import pallas as pl
from jax.experimental.pallas import tpu as pltpu
from jax.experimental.pallas import tpu_sc as plsc

C = 32
NB = 8
SH = 9
R_MAX = 5.0
AVG = 32.0
S3 = float(np.sqrt(3.0))
S5 = float(np.sqrt(5.0))
S15 = float(np.sqrt(15.0))
PREF = float(np.sqrt(2.0 / R_MAX))
KN = (np.arange(1, NB + 1, dtype=np.float32) * np.pi / R_MAX).reshape(1, NB)

BE = 1280
BN = 2000

NW = 32
KCH = 80


def _silu(x):
    return x * jax.nn.sigmoid(x)


def _dsilu(x):
    s = jax.nn.sigmoid(x)
    return s * (1.0 + x * (1.0 - s))


def _dot(a, b):
    return jnp.dot(a, b, preferred_element_type=jnp.float32,
                   precision=lax.Precision.HIGHEST)


def _dotd(a, b):
    return jnp.dot(a, b, preferred_element_type=jnp.float32)


def _sc_gather(table, idx):
    n, d = table.shape
    e = idx.shape[0]
    per_w = e // NW
    iters = per_w // KCH
    mesh = plsc.VectorSubcoreMesh(core_axis_name="c", subcore_axis_name="s")

    @functools.partial(
        pl.kernel, mesh=mesh,
        compiler_params=pltpu.CompilerParams(use_tc_tiling_on_sc=False),
        out_type=jax.ShapeDtypeStruct((e, d), jnp.float32),
        scratch_types=[
            pltpu.VMEM((iters, KCH), jnp.int32),
            pltpu.VMEM((KCH, d), jnp.float32),
            pltpu.SemaphoreType.DMA,
        ],
    )
    def k(tab, ix, out, idx_all, rows_v, sem):
        cid = lax.axis_index("c")
        sid = lax.axis_index("s")
        wid = sid * 2 + cid
        base = wid * per_w
        pltpu.sync_copy(ix.at[pl.ds(wid * iters, iters)], idx_all)

        def body(j, carry):
            off = base + j * KCH
            pltpu.async_copy(tab.at[idx_all.at[j]], rows_v, sem).wait()
            pltpu.sync_copy(rows_v, out.at[pl.ds(off, KCH)])
            return carry

        lax.fori_loop(0, iters, body, 0)

    return k(table, idx.reshape(e // KCH, KCH))


def _sc_scatter_add(payload, idx, n, zeros_nd):
    e, d = payload.shape
    per_w = e // NW
    iters = per_w // KCH
    rows_per_tile = n // 16
    mesh = plsc.VectorSubcoreMesh(core_axis_name="c", subcore_axis_name="s")

    @functools.partial(
        pl.kernel, mesh=mesh,
        compiler_params=pltpu.CompilerParams(use_tc_tiling_on_sc=False),
        out_type=jax.ShapeDtypeStruct((2, n, d), jnp.float32),
        scratch_types=[
            pltpu.VMEM((1, KCH), jnp.int32),
            pltpu.VMEM((KCH, d), jnp.float32),
            pltpu.VMEM_SHARED((n, d), jnp.float32),
        ],
    )
    def k(pay, ix, zr, out, idx_v, pay_v, acc):
        cid = lax.axis_index("c")
        sid = lax.axis_index("s")
        wid = sid * 2 + cid
        base = wid * per_w
        r0 = sid * rows_per_tile
        pltpu.sync_copy(zr.at[pl.ds(r0, rows_per_tile)],
                        acc.at[pl.ds(r0, rows_per_tile)])
        plsc.subcore_barrier()

        def body(j, carry):
            off = base + j * KCH
            row = wid * iters + j
            pltpu.sync_copy(ix.at[pl.ds(row, 1)], idx_v)
            pltpu.sync_copy(pay.at[pl.ds(off, KCH)], pay_v)
            pltpu.sync_copy(pay_v, acc.at[idx_v.at[0]], add=True)
            return carry

        lax.fori_loop(0, iters, body, 0)
        plsc.subcore_barrier()
        pltpu.sync_copy(acc.at[pl.ds(r0, rows_per_tile)],
                        out.at[cid, pl.ds(r0, rows_per_tile)])

    return k(payload, idx.reshape(e // KCH, KCH), zeros_nd)


def _mlp_fwd(f, w1, w2, w3):
    r1 = _dotd(f, w1)
    a1 = _silu(r1)
    r2 = _dotd(a1, w2)
    a2 = _silu(r2)
    return r1, r2, _dotd(a2, w3)


def _mlp_bwd(gr, f, w1, w2, r1, r2, w3t, w2t, w1t):
    ga2 = _dotd(gr, w3t)
    gr2 = ga2 * _dsilu(r2)
    ga1 = _dotd(gr2, w2t)
    gr1 = ga1 * _dsilu(r1)
    return _dotd(gr1, w1t)


def _geom_edge0_body(ps_ref, pd_ref, w1, w2, w3, wm, we, wv,
                     geo_ref, scat_ref):
    ps = ps_ref[...]
    pd = pd_ref[...]
    v = pd[:, 0:3] - ps[:, 0:3]
    ln = jnp.sqrt(jnp.sum(v * v, axis=1, keepdims=True) + 1e-12)
    u = v / ln
    x = u[:, 0:1]; y = u[:, 1:2]; z = u[:, 2:3]
    one = jnp.ones_like(x)
    a = jnp.concatenate([
        one, S3 * x, S3 * y, S3 * z,
        S15 * x * y, S15 * y * z,
        0.5 * S5 * (2.0 * z * z - x * x - y * y),
        S15 * x * z, 0.5 * S15 * (x * x - y * y)], axis=1)
    kn = (lax.broadcasted_iota(jnp.int32, (1, NB), 1).astype(jnp.float32) + 1.0) * (np.pi / R_MAX)
    sin_t = jnp.sin(ln * kn)
    bes = PREF * sin_t / ln
    uu = ln / R_MAX
    msk = (uu < 1.0).astype(jnp.float32)
    u4 = uu * uu * uu * uu * uu
    cut = (1.0 - 21.0 * u4 + 35.0 * u4 * uu - 15.0 * u4 * uu * uu) * msk
    f = bes * cut
    pad = jnp.zeros((ps.shape[0], 11), jnp.float32)
    geo_ref[...] = jnp.concatenate([u, ln, a, f, pad], axis=1)

    _, _, r = _mlp_fwd(f, w1[...], w2[...], w3[...])
    msg = r * we[...]
    acc = jnp.zeros((ps.shape[0], C), jnp.float32)
    wmv = wm[...]
    for s in range(SH):
        acc = acc + _dot(msg * a[:, s:s + 1], wmv[s])
    vc = _dot(msg, wv[...]) * a[:, 1:4]
    pad1 = jnp.zeros((ps.shape[0], 5), jnp.float32)
    scat_ref[...] = jnp.concatenate([acc, vc, pad1], axis=1)


def _node0_body(agg_ref, we, wsc, wp, wpost, wread, h1_ref, m0_ref, ev_ref):
    p = agg_ref[0] + agg_ref[1]
    m0 = p[:, 0:C] * (1.0 / AVG)
    wpv = wp[...]
    w0 = wpv[0:1, :]; w1 = wpv[1:2, :]; w2 = wpv[2:3, :]
    feats = w0 * m0 + w1 * m0 * m0 + w2 * m0 * m0 * m0
    scrow = _dotd(we[...], wsc[...])
    h1 = _dotd(feats, wpost[...]) + scrow
    h1_ref[...] = h1
    m0_ref[...] = m0
    en = _dotd(h1, wread[...])
    ev_ref[...] = jnp.concatenate([en, p[:, C:C + 3]], axis=1)


def _edge1_body(geo_ref, g1_ref, w1, w2, w3, wm, wv, scat_ref):
    geo = geo_ref[...]
    a = geo[:, 4:13]
    f = geo[:, 13:21]
    _, _, r = _mlp_fwd(f, w1[...], w2[...], w3[...])
    g1 = g1_ref[...]
    msg = g1 * r
    acc = jnp.zeros((geo.shape[0], C), jnp.float32)
    wmv = wm[...]
    for s in range(SH):
        acc = acc + _dot(msg * a[:, s:s + 1], wmv[s])
    vc = _dot(msg, wv[...]) * a[:, 1:4]
    pad1 = jnp.zeros((geo.shape[0], 5), jnp.float32)
    scat_ref[...] = jnp.concatenate([acc, vc, pad1], axis=1)


def _node1_body(agg_ref, h1_ref, ev_ref, wsc, wp, wpost, wmlp1, wmlp2t,
                wsct, wpostt, wmlp1t, wreadt,
                gm1_ref, gh1a_ref, en_ref, vec_ref):
    p = agg_ref[0] + agg_ref[1]
    h1 = h1_ref[...]
    ev = ev_ref[...]
    m1 = p[:, 0:C] * (1.0 / AVG)
    wpv = wp[...]
    w0 = wpv[0:1, :]; w1 = wpv[1:2, :]; w2 = wpv[2:3, :]
    feats = w0 * m1 + w1 * m1 * m1 + w2 * m1 * m1 * m1
    h2 = _dotd(feats, wpost[...]) + _dotd(h1, wsc[...])
    u2 = _dotd(h2, wmlp1[...])
    en_b = _dotd(_silu(u2), jnp.transpose(wmlp2t[...]))
    en_ref[...] = ev[:, 0:1] + en_b
    vec_ref[...] = (ev[:, 1:4] + p[:, C:C + 3]) * (1.0 / AVG)
    gh2 = _dotd(_dsilu(u2) * wmlp2t[...], wmlp1t[...])
    gfeats = _dotd(gh2, wpostt[...])
    gm1 = gfeats * (w0 + 2.0 * w1 * m1 + 3.0 * w2 * m1 * m1)
    gm1_ref[...] = gm1 * (1.0 / AVG)
    gh1a_ref[...] = _dotd(gh2, wsct[...]) + wreadt[...]


def _edge1_bwd_body(geo_ref, g1_ref, gg1_ref, w1, w2, w3, wmt,
                    w3t, w2t, w1t, gsrc_ref, gaf_ref):
    geo = geo_ref[...]
    a = geo[:, 4:13]
    f = geo[:, 13:21]
    w1v = w1[...]; w2v = w2[...]
    r1, r2, r = _mlp_fwd(f, w1v, w2v, w3[...])
    g1 = g1_ref[...]
    msg = g1 * r
    gg1 = gg1_ref[...]
    wmtv = wmt[...]
    gmsg = jnp.zeros((geo.shape[0], C), jnp.float32)
    gas = []
    for s in range(SH):
        ys = _dot(gg1, wmtv[s])
        gmsg = gmsg + a[:, s:s + 1] * ys
        gas.append(jnp.sum(msg * ys, axis=1, keepdims=True))
    ga = jnp.concatenate(gas, axis=1)
    gsrc_ref[...] = gmsg * r
    gr = gmsg * g1
    gf = _mlp_bwd(gr, f, w1v, w2v, r1, r2, w3t[...], w2t[...], w1t[...])
    pad = jnp.zeros((geo.shape[0], 7), jnp.float32)
    gaf_ref[...] = jnp.concatenate([ga, gf, pad], axis=1)


def _node0_bwd_body(gh1a_ref, ghb_ref, m0_ref, wpostt, wp, gm0_ref):
    gh1 = gh1a_ref[...] + ghb_ref[0] + ghb_ref[1]
    gfeats = _dotd(gh1, wpostt[...])
    m0 = m0_ref[...]
    wpv = wp[...]
    w0 = wpv[0:1, :]; w1 = wpv[1:2, :]; w2 = wpv[2:3, :]
    gm0 = gfeats * (w0 + 2.0 * w1 * m0 + 3.0 * w2 * m0 * m0)
    gm0_ref[...] = gm0 * (1.0 / AVG)


def _edge0_bwd_body(geo_ref, gg0_ref, gaf_ref, w1, w2, w3, wmt, we,
                    w3t, w2t, w1t, gv_ref):
    geo = geo_ref[...]
    u = geo[:, 0:3]
    ln = geo[:, 3:4]
    a = geo[:, 4:13]
    f = geo[:, 13:21]
    w1v = w1[...]; w2v = w2[...]
    r1, r2, r = _mlp_fwd(f, w1v, w2v, w3[...])
    msg = r * we[...]
    gg0 = gg0_ref[...]
    gaf = gaf_ref[...]
    wmtv = wmt[...]
    gmsg = jnp.zeros((geo.shape[0], C), jnp.float32)
    gas = []
    for s in range(SH):
        ys = _dot(gg0, wmtv[s])
        gmsg = gmsg + a[:, s:s + 1] * ys
        gas.append(jnp.sum(msg * ys, axis=1, keepdims=True))
    ga = jnp.concatenate(gas, axis=1) + gaf[:, 0:SH]
    gr = gmsg * we[...]
    gf = _mlp_bwd(gr, f, w1v, w2v, r1, r2, w3t[...], w2t[...], w1t[...])
    gf = gf + gaf[:, SH:SH + NB]
    kn = (lax.broadcasted_iota(jnp.int32, (1, NB), 1).astype(jnp.float32) + 1.0) * (np.pi / R_MAX)
    sin_t = jnp.sin(ln * kn)
    cos_t = jnp.cos(ln * kn)
    bes = PREF * sin_t / ln
    uu = ln / R_MAX
    msk = (uu < 1.0).astype(jnp.float32)
    u4 = uu * uu * uu * uu
    u5 = u4 * uu
    cut = (1.0 - 21.0 * u5 + 35.0 * u5 * uu - 15.0 * u5 * uu * uu) * msk
    dbes = PREF * (kn * cos_t / ln - sin_t / (ln * ln))
    dcut = (-105.0 * u4 + 210.0 * u5 - 105.0 * u5 * uu) * msk * (1.0 / R_MAX)
    glen = jnp.sum(gf * (dbes * cut + bes * dcut), axis=1, keepdims=True)
    x = u[:, 0:1]; y = u[:, 1:2]; z = u[:, 2:3]
    ga1 = ga[:, 1:2]; ga2 = ga[:, 2:3]; ga3 = ga[:, 3:4]
    ga4 = ga[:, 4:5]; ga5 = ga[:, 5:6]; ga6 = ga[:, 6:7]
    ga7 = ga[:, 7:8]; ga8 = ga[:, 8:9]
    gx = S3 * ga1 + S15 * (y * ga4 + z * ga7) - S5 * x * ga6 + S15 * x * ga8
    gy = S3 * ga2 + S15 * (x * ga4 + z * ga5) - S5 * y * ga6 - S15 * y * ga8
    gz = S3 * ga3 + S15 * (y * ga5 + x * ga7) + 2.0 * S5 * z * ga6
    gu = jnp.concatenate([gx, gy, gz], axis=1)
    gv = glen * u + (gu - u * jnp.sum(u * gu, axis=1, keepdims=True)) / ln
    pad1 = jnp.zeros((geo.shape[0], 5), jnp.float32)
    gv_ref[...] = jnp.concatenate([gv, pad1], axis=1)


def _forces_body(fs_ref, fd_ref, out_ref):
    out_ref[...] = (fs_ref[0] + fs_ref[1] - fd_ref[0] - fd_ref[1])[:, 0:3]


def _energy_body(en_ref, na_ref, ae_ref, out_ref):
    out_ref[...] = jnp.sum(en_ref[...] + na_ref[...] * ae_ref[0, 0],
                           axis=1, keepdims=True)


def _wspec(shape):
    nd = len(shape)
    return pl.BlockSpec(shape, lambda i, _nd=nd: (0,) * _nd)


def kernel(positions, node_attrs, edge_index, shifts, cell, batch, ptr, params):
    n = positions.shape[0]
    e = edge_index.shape[1]
    g = cell.shape[0]
    ge = e // BE
    gn = n // BN
    f32 = jnp.float32

    src = edge_index[0].astype(jnp.int32)
    dst = edge_index[1].astype(jnp.int32)
    pos8 = jnp.concatenate([positions.astype(f32),
                            jnp.zeros((n, 5), f32)], axis=1)

    p = {k: v.astype(f32) for k, v in params.items()}
    wm = [p['W_msg_%d' % i].reshape(C, SH, C).transpose(1, 0, 2)
          for i in range(2)]
    wmt = [jnp.transpose(w, (0, 2, 1)) for w in wm]
    z40 = jnp.zeros((n, 40), f32)
    z32 = jnp.zeros((n, 32), f32)
    z8 = jnp.zeros((n, 8), f32)

    espec = lambda d: pl.BlockSpec((BE, d), lambda i: (i, 0))
    nspec = lambda d: pl.BlockSpec((BN, d), lambda i: (i, 0))
    n2spec = lambda d: pl.BlockSpec((2, BN, d), lambda i: (0, i, 0))

    ps = _sc_gather(pos8, src)
    pd = _sc_gather(pos8, dst)

    geo, scat0 = pl.pallas_call(
        _geom_edge0_body,
        grid=(ge,),
        in_specs=[espec(8), espec(8), _wspec((NB, 64)), _wspec((64, 64)),
                  _wspec((64, C)), _wspec((SH, C, C)), _wspec((1, C)),
                  _wspec((C, 1))],
        out_specs=[espec(32), espec(40)],
        out_shape=[jax.ShapeDtypeStruct((e, 32), f32),
                   jax.ShapeDtypeStruct((e, 40), f32)],
    )(ps, pd, p['W_r1_0'], p['W_r2_0'], p['W_r3_0'], wm[0],
      p['W_embed'], p['W_vec_0'])

    agg0 = _sc_scatter_add(scat0, dst, n, z40)

    h1, m0, ev0 = pl.pallas_call(
        _node0_body,
        grid=(gn,),
        in_specs=[n2spec(40), _wspec((1, C)), _wspec((C, C)),
                  _wspec((3, C)), _wspec((C, C)), _wspec((C, 1))],
        out_specs=[nspec(32), nspec(32), nspec(4)],
        out_shape=[jax.ShapeDtypeStruct((n, 32), f32),
                   jax.ShapeDtypeStruct((n, 32), f32),
                   jax.ShapeDtypeStruct((n, 4), f32)],
    )(agg0, p['W_embed'], p['W_sc_0'], p['W_prod_0'], p['W_post_0'],
      p['W_read_0'])

    g1 = _sc_gather(h1, src)

    scat1 = pl.pallas_call(
        _edge1_body,
        grid=(ge,),
        in_specs=[espec(32), espec(32), _wspec((NB, 64)), _wspec((64, 64)),
                  _wspec((64, C)), _wspec((SH, C, C)), _wspec((C, 1))],
        out_specs=espec(40),
        out_shape=jax.ShapeDtypeStruct((e, 40), f32),
    )(geo, g1, p['W_r1_1'], p['W_r2_1'], p['W_r3_1'], wm[1], p['W_vec_1'])

    agg1 = _sc_scatter_add(scat1, dst, n, z40)

    gm1, gh1a, en, vec_out = pl.pallas_call(
        _node1_body,
        grid=(gn,),
        in_specs=[n2spec(40), nspec(32), nspec(4), _wspec((C, C)),
                  _wspec((3, C)), _wspec((C, C)), _wspec((C, 16)),
                  _wspec((1, 16)), _wspec((C, C)), _wspec((C, C)),
                  _wspec((16, C)), _wspec((1, C))],
        out_specs=[nspec(32), nspec(32), nspec(1), nspec(3)],
        out_shape=[jax.ShapeDtypeStruct((n, 32), f32),
                   jax.ShapeDtypeStruct((n, 32), f32),
                   jax.ShapeDtypeStruct((n, 1), f32),
                   jax.ShapeDtypeStruct((n, 3), f32)],
    )(agg1, h1, ev0, p['W_sc_1'], p['W_prod_1'], p['W_post_1'], p['W_mlp1'],
      p['W_mlp2'].T, p['W_sc_1'].T, p['W_post_1'].T, p['W_mlp1'].T,
      p['W_read_0'].T)

    gg1 = _sc_gather(gm1, dst)

    gsrc, gaf1 = pl.pallas_call(
        _edge1_bwd_body,
        grid=(ge,),
        in_specs=[espec(32), espec(32), espec(32), _wspec((NB, 64)),
                  _wspec((64, 64)), _wspec((64, C)), _wspec((SH, C, C)),
                  _wspec((C, 64)), _wspec((64, 64)), _wspec((64, NB))],
        out_specs=[espec(32), espec(24)],
        out_shape=[jax.ShapeDtypeStruct((e, 32), f32),
                   jax.ShapeDtypeStruct((e, 24), f32)],
    )(geo, g1, gg1, p['W_r1_1'], p['W_r2_1'], p['W_r3_1'], wmt[1],
      p['W_r3_1'].T, p['W_r2_1'].T, p['W_r1_1'].T)

    ghb = _sc_scatter_add(gsrc, src, n, z32)

    gm0 = pl.pallas_call(
        _node0_bwd_body,
        grid=(gn,),
        in_specs=[nspec(32), n2spec(32), nspec(32), _wspec((C, C)),
                  _wspec((3, C))],
        out_specs=nspec(32),
        out_shape=jax.ShapeDtypeStruct((n, 32), f32),
    )(gh1a, ghb, m0, p['W_post_0'].T, p['W_prod_0'])

    gg0 = _sc_gather(gm0, dst)

    gv = pl.pallas_call(
        _edge0_bwd_body,
        grid=(ge,),
        in_specs=[espec(32), espec(32), espec(24), _wspec((NB, 64)),
                  _wspec((64, 64)), _wspec((64, C)), _wspec((SH, C, C)),
                  _wspec((1, C)), _wspec((C, 64)), _wspec((64, 64)),
                  _wspec((64, NB))],
        out_specs=espec(8),
        out_shape=jax.ShapeDtypeStruct((e, 8), f32),
    )(geo, gg0, gaf1, p['W_r1_0'], p['W_r2_0'], p['W_r3_0'], wmt[0],
      p['W_embed'], p['W_r3_0'].T, p['W_r2_0'].T, p['W_r1_0'].T)

    fs = _sc_scatter_add(gv, src, n, z8)
    fd = _sc_scatter_add(gv, dst, n, z8)

    forces = pl.pallas_call(
        _forces_body,
        grid=(gn,),
        in_specs=[n2spec(8), n2spec(8)],
        out_specs=nspec(3),
        out_shape=jax.ShapeDtypeStruct((n, 3), f32),
    )(fs, fd)

    npg = n // g
    te = pl.pallas_call(
        _energy_body,
        grid=(1,),
        in_specs=[pl.BlockSpec((g, npg), lambda i: (0, 0)),
                  pl.BlockSpec((g, npg), lambda i: (0, 0)),
                  pl.BlockSpec((1, 1), lambda i: (0, 0))],
        out_specs=pl.BlockSpec((g, 1), lambda i: (0, 0)),
        out_shape=jax.ShapeDtypeStruct((g, 1), f32),
    )(en.reshape(g, npg), node_attrs.astype(f32).reshape(g, npg),
      p['atomic_energies'].reshape(1, 1))

    return te[:, 0], forces, vec_out
